# Initial kernel scaffold; baseline (speedup 1.0000x reference)
import jax
import jax.numpy as jnp
from jax.experimental import pallas as pl


def kernel(x, W):
    raise NotImplementedError("write your pallas kernel here")


# speedup vs baseline: 5.6914x; 5.6914x over previous
import functools

import jax
import jax.numpy as jnp
from jax import lax
from jax.experimental import pallas as pl
from jax.experimental.pallas import tpu as pltpu
from jax.experimental.pallas import tpu_sc as plsc

_B, _L, _D = 16384, 200, 8
_N = _B * _L
_NW = 32
_PER_W = _N // _NW
_CHUNK = 2048
_NCHUNK = _PER_W // _CHUNK
_NVEC = _CHUNK // 16

_mesh = plsc.VectorSubcoreMesh(core_axis_name="c", subcore_axis_name="s")


@functools.partial(
    pl.kernel,
    mesh=_mesh,
    out_type=jax.ShapeDtypeStruct((_N * _D,), jnp.float32),
    scratch_types=[
        pltpu.VMEM((264,), jnp.float32),
        pltpu.VMEM((_CHUNK,), jnp.int32),
        pltpu.VMEM((_CHUNK * _D,), jnp.float32),
    ],
    compiler_params=pltpu.CompilerParams(needs_layout_passes=False),
)
def _lookup(table_hbm, idx_hbm, out_hbm, tflat_v, idx_v, rows_v):
    wid = lax.axis_index("s") * 2 + lax.axis_index("c")
    base = wid * _PER_W

    pltpu.sync_copy(table_hbm, tflat_v)

    iota = lax.iota(jnp.int32, 16)
    st_base = iota * _D

    def chunk_body(g, carry):
        off = base + g * _CHUNK
        pltpu.sync_copy(idx_hbm.at[pl.ds(off, _CHUNK)], idx_v)

        def vec_body(i, carry2):
            xv8 = idx_v[pl.ds(i * 16, 16)] * _D
            st = st_base + i * (16 * _D)
            for d in range(_D):
                vals = plsc.load_gather(tflat_v, [xv8 + d])
                plsc.store_scatter(rows_v, [st + d], vals)
            return carry2

        lax.fori_loop(0, _NVEC, vec_body, 0)
        pltpu.sync_copy(rows_v, out_hbm.at[pl.ds(off * _D, _CHUNK * _D)])
        return carry

    lax.fori_loop(0, _NCHUNK, chunk_body, 0)


def kernel(x, W):
    out = _lookup(W.reshape(33 * _D), x.reshape(_N))
    return out.reshape(_B, _L, _D)


# speedup vs baseline: 189.0567x; 33.2179x over previous
import functools

import jax
import jax.numpy as jnp
from jax import lax
from jax.experimental import pallas as pl
from jax.experimental.pallas import tpu as pltpu
from jax.experimental.pallas import tpu_sc as plsc

_B, _L, _D = 16384, 200, 8
_NW = 32
_BS = _B // _NW
_LC = 8
_NCH = _L // _LC
_NBV = _BS // 16

_mesh = plsc.VectorSubcoreMesh(core_axis_name="c", subcore_axis_name="s")


@functools.partial(
    pl.kernel,
    mesh=_mesh,
    out_type=jax.ShapeDtypeStruct((_L * _D, _B), jnp.float32),
    scratch_types=[
        pltpu.VMEM((_D, 33), jnp.float32),
        pltpu.VMEM((2, _LC, _BS), jnp.int32),
        pltpu.VMEM((2, _LC * _D, _BS), jnp.float32),
        pltpu.SemaphoreType.DMA,
        pltpu.SemaphoreType.DMA,
    ],
    compiler_params=pltpu.CompilerParams(needs_layout_passes=False),
)
def _lookup(xt_hbm, wt_hbm, out_hbm, wt_v, idx_v, stage_v, isem, osem):
    wid = lax.axis_index("s") * 2 + lax.axis_index("c")
    b0 = wid * _BS

    pltpu.sync_copy(wt_hbm, wt_v)

    dsplat = [jnp.full((16,), d, jnp.int32) for d in range(_D)]

    def issue_in(ci, buf):
        pltpu.async_copy(
            xt_hbm.at[pl.ds(ci * _LC, _LC), pl.ds(b0, _BS)],
            idx_v.at[buf],
            isem,
        )

    def wait_in(buf):
        pltpu.make_async_copy(
            xt_hbm.at[pl.ds(0, _LC), pl.ds(b0, _BS)], idx_v.at[buf], isem
        ).wait()

    def issue_out(ci, buf):
        pltpu.async_copy(
            stage_v.at[buf],
            out_hbm.at[pl.ds(ci * _LC * _D, _LC * _D), pl.ds(b0, _BS)],
            osem,
        )

    def wait_out(buf):
        pltpu.make_async_copy(
            stage_v.at[buf],
            out_hbm.at[pl.ds(0, _LC * _D), pl.ds(b0, _BS)],
            osem,
        ).wait()

    issue_in(0, 0)

    def chunk(ci, carry):
        buf = lax.rem(ci, 2)

        @pl.when(ci + 1 < _NCH)
        def _prefetch():
            issue_in(ci + 1, 1 - buf)

        wait_in(buf)

        @pl.when(ci >= 2)
        def _wait_prev_out():
            wait_out(buf)

        for l in range(_LC):
            @plsc.parallel_loop(0, _NBV, unroll=2)
            def _bv(k):
                bo = k * 16
                xv = idx_v[buf, l, pl.ds(bo, 16)]
                for d in range(_D):
                    val = plsc.load_gather(wt_v, [dsplat[d], xv])
                    stage_v[buf, l * _D + d, pl.ds(bo, 16)] = val

        issue_out(ci, buf)
        return carry

    lax.fori_loop(0, _NCH, chunk, 0)

    wait_out(lax.rem(_NCH - 2, 2))
    wait_out(lax.rem(_NCH - 1, 2))


def kernel(x, W):
    out2 = _lookup(x.T, W.T)
    return out2.reshape(_L, _D, _B).transpose(2, 0, 1)


# speedup vs baseline: 189.4781x; 1.0022x over previous
import functools

import jax
import jax.numpy as jnp
from jax import lax
from jax.experimental import pallas as pl
from jax.experimental.pallas import tpu as pltpu
from jax.experimental.pallas import tpu_sc as plsc

_B, _L, _D = 16384, 200, 8
_NW = 32
_BS = _B // _NW
_LC = 8
_NCH = _L // _LC
_NBV = _BS // 16

_mesh = plsc.VectorSubcoreMesh(core_axis_name="c", subcore_axis_name="s")


@functools.partial(
    pl.kernel,
    mesh=_mesh,
    out_type=jax.ShapeDtypeStruct((_L * _D, _B), jnp.float32),
    scratch_types=[
        pltpu.VMEM((_D, 33), jnp.float32),
        pltpu.VMEM((2, _LC, _BS), jnp.int32),
        pltpu.VMEM((2, _LC * _D, _BS), jnp.float32),
        pltpu.SemaphoreType.DMA,
        pltpu.SemaphoreType.DMA,
    ],
    compiler_params=pltpu.CompilerParams(
        needs_layout_passes=False, disable_bounds_checks=True
    ),
)
def _lookup(xt_hbm, wt_hbm, out_hbm, wt_v, idx_v, stage_v, isem, osem):
    wid = lax.axis_index("s") * 2 + lax.axis_index("c")
    b0 = wid * _BS

    pltpu.sync_copy(wt_hbm, wt_v)

    dsplat = [jnp.full((16,), d, jnp.int32) for d in range(_D)]

    def issue_in(ci, buf):
        pltpu.async_copy(
            xt_hbm.at[pl.ds(ci * _LC, _LC), pl.ds(b0, _BS)],
            idx_v.at[buf],
            isem,
        )

    def wait_in(buf):
        pltpu.make_async_copy(
            xt_hbm.at[pl.ds(0, _LC), pl.ds(b0, _BS)], idx_v.at[buf], isem
        ).wait()

    def issue_out(ci, buf):
        pltpu.async_copy(
            stage_v.at[buf],
            out_hbm.at[pl.ds(ci * _LC * _D, _LC * _D), pl.ds(b0, _BS)],
            osem,
        )

    def wait_out(buf):
        pltpu.make_async_copy(
            stage_v.at[buf],
            out_hbm.at[pl.ds(0, _LC * _D), pl.ds(b0, _BS)],
            osem,
        ).wait()

    issue_in(0, 0)

    def chunk(ci, carry):
        buf = lax.rem(ci, 2)

        @pl.when(ci + 1 < _NCH)
        def _prefetch():
            issue_in(ci + 1, 1 - buf)

        wait_in(buf)

        @pl.when(ci >= 2)
        def _wait_prev_out():
            wait_out(buf)

        for l in range(_LC):
            @plsc.parallel_loop(0, _NBV, unroll=4)
            def _bv(k):
                bo = k * 16
                xv = idx_v[buf, l, pl.ds(bo, 16)]
                for d in range(_D):
                    val = plsc.load_gather(wt_v, [dsplat[d], xv])
                    stage_v[buf, l * _D + d, pl.ds(bo, 16)] = val

        issue_out(ci, buf)
        return carry

    lax.fori_loop(0, _NCH, chunk, 0)

    wait_out(lax.rem(_NCH - 2, 2))
    wait_out(lax.rem(_NCH - 1, 2))


def kernel(x, W):
    out2 = _lookup(x.T, W.T)
    return out2.reshape(_L, _D, _B).transpose(2, 0, 1)
